# baseline (speedup 1.0000x reference)
import functools
import jax
import jax.numpy as jnp
import numpy as np
from jax.experimental import pallas as pl
from jax.experimental.pallas import tpu as pltpu

HID = 256
HEADS = 8
HD = 32
FFN2 = 512
NUM_REL = 4
ID_EMB = 32
NUM_GRAPHS = 64
EPS = 1e-5

BN = 2000
BE = 4000


def _silu(x):
    return x * (1.0 / (1.0 + jnp.exp(-x)))


def _ln(x, s, b):
    m = jnp.mean(x, axis=-1, keepdims=True)
    v = jnp.mean((x - m) ** 2, axis=-1, keepdims=True)
    return (x - m) * jax.lax.rsqrt(v + EPS) * s + b


def _in_proj_body(xid_ref, w_ref, b_ref, o_ref):
    acc = jnp.dot(xid_ref[...], w_ref[...], preferred_element_type=jnp.float32)
    o_ref[...] = _silu(acc + b_ref[...])


def _node_att_body(h_ref, w_ref, asd_m_ref, hw_ref, asd_ref):
    hw = jnp.dot(h_ref[...], w_ref[...], preferred_element_type=jnp.float32)
    hw_ref[...] = hw
    asd_ref[...] = jnp.dot(hw, asd_m_ref[...], preferred_element_type=jnp.float32)


def _edge_body(ea_ref, et_ref, rel_ref, wea_ref, wrel_ref, wa_ref, e_ref, ae_ref):
    et = et_ref[...]
    rel = jnp.zeros((et.shape[0], 8), jnp.float32)
    for r in range(NUM_REL):
        rel = rel + jnp.where(et == r, 1.0, 0.0) * rel_ref[r:r + 1, :]
    ea = ea_ref[...]
    e = (jnp.dot(ea, wea_ref[...], preferred_element_type=jnp.float32)
         + jnp.dot(rel, wrel_ref[...], preferred_element_type=jnp.float32))
    e_ref[...] = e
    ae_ref[...] = jnp.dot(e, wa_ref[...], preferred_element_type=jnp.float32)


def _ffn_body(h_ref, agg_ref, wo_ref, bo_ref, l1s_ref, l1b_ref,
              w1_ref, b1_ref, w2_ref, b2_ref, l2s_ref, l2b_ref, o_ref):
    t = h_ref[...] + jnp.dot(agg_ref[...], wo_ref[...],
                             preferred_element_type=jnp.float32) + bo_ref[...]
    h1 = _ln(t, l1s_ref[...], l1b_ref[...])
    u = _silu(jnp.dot(h1, w1_ref[...], preferred_element_type=jnp.float32) + b1_ref[...])
    f = jnp.dot(u, w2_ref[...], preferred_element_type=jnp.float32) + b2_ref[...]
    o_ref[...] = _ln(h1 + f, l2s_ref[...], l2b_ref[...])


def _final_ln_body(g_ref, s_ref, b_ref, o_ref):
    o_ref[...] = _ln(g_ref[...], s_ref[...], b_ref[...])


def _full(shape):
    nd = len(shape)
    return pl.BlockSpec(shape, lambda i: (0,) * nd)


def _rows(bshape):
    nd = len(bshape)
    return pl.BlockSpec(bshape, lambda i: (i,) + (0,) * (nd - 1))


def _in_proj(xid, w, b):
    n = xid.shape[0]
    return pl.pallas_call(
        _in_proj_body,
        grid=(n // BN,),
        in_specs=[_rows((BN, xid.shape[1])), _full(w.shape), _full(b.shape)],
        out_specs=_rows((BN, HID)),
        out_shape=jax.ShapeDtypeStruct((n, HID), jnp.float32),
    )(xid, w, b)


def _node_att(h, w, asd_m):
    n = h.shape[0]
    return pl.pallas_call(
        _node_att_body,
        grid=(n // BN,),
        in_specs=[_rows((BN, HID)), _full(w.shape), _full(asd_m.shape)],
        out_specs=[_rows((BN, HID)), _rows((BN, 16))],
        out_shape=[jax.ShapeDtypeStruct((n, HID), jnp.float32),
                   jax.ShapeDtypeStruct((n, 16), jnp.float32)],
    )(h, w, asd_m)


def _edge_stage(ea, et2, rel_emb, wea, wrel, wa):
    e_cnt = ea.shape[0]
    return pl.pallas_call(
        _edge_body,
        grid=(e_cnt // BE,),
        in_specs=[_rows((BE, ea.shape[1])), _rows((BE, 1)), _full(rel_emb.shape),
                  _full(wea.shape), _full(wrel.shape), _full(wa.shape)],
        out_specs=[_rows((BE, HID)), _rows((BE, 8))],
        out_shape=[jax.ShapeDtypeStruct((e_cnt, HID), jnp.float32),
                   jax.ShapeDtypeStruct((e_cnt, 8), jnp.float32)],
    )(ea, et2, rel_emb, wea, wrel, wa)


def _ffn(h, agg, p):
    n = h.shape[0]
    args = [h, agg, p['W_out'], p['b_out'].reshape(1, HID),
            p['ln1_s'].reshape(1, HID), p['ln1_b'].reshape(1, HID),
            p['W1'], p['b1'].reshape(1, FFN2),
            p['W2'], p['b2'].reshape(1, HID),
            p['ln2_s'].reshape(1, HID), p['ln2_b'].reshape(1, HID)]
    in_specs = [_rows((BN, HID)), _rows((BN, HID))] + [_full(a.shape) for a in args[2:]]
    return pl.pallas_call(
        _ffn_body,
        grid=(n // BN,),
        in_specs=in_specs,
        out_specs=_rows((BN, HID)),
        out_shape=jax.ShapeDtypeStruct((n, HID), jnp.float32),
    )(*args)


def _final_ln(g, s, b):
    return pl.pallas_call(
        _final_ln_body,
        grid=(1,),
        in_specs=[_full(g.shape), _full(s.shape), _full(b.shape)],
        out_specs=_full(g.shape),
        out_shape=jax.ShapeDtypeStruct(g.shape, jnp.float32),
    )(g, s, b)


def kernel(x, id_token, edge_index, edge_attr, edge_type, batch, params):
    n = x.shape[0]
    src, dst = edge_index[0], edge_index[1]

    def _att_mat(att):
        m = np.zeros((HID, HEADS), np.float32)
        mask = jnp.asarray(
            np.stack([np.repeat(np.eye(HEADS, dtype=np.float32)[h], HD)
                      for h in range(HEADS)], axis=1).reshape(HID, HEADS))
        return mask * att.reshape(HID, 1)

    id_emb = params['id_embedding'][id_token]
    xid = jnp.concatenate([x, id_emb], axis=1)
    h = _in_proj(xid, params['in_W'], params['in_b'].reshape(1, HID))

    et2 = edge_type.reshape(-1, 1).astype(jnp.int32)

    for p in params['blocks']:
        asd_m = jnp.concatenate([_att_mat(p['att_src']), _att_mat(p['att_dst'])], axis=1)
        hw, asd = _node_att(h, p['W'], asd_m)
        wa_edge = _att_mat(p['att_edge'])
        e, a_edge = _edge_stage(edge_attr, et2, p['rel_emb'],
                                p['W_edge'][:16], p['W_edge'][16:], wa_edge)

        logits = jax.nn.leaky_relu(asd[src, 0:8] + asd[dst, 8:16] + a_edge, 0.2)
        ex = jnp.exp(logits)
        den = jax.ops.segment_sum(ex, dst, num_segments=n)
        alpha = ex / (den[dst] + 1e-16)
        msg = (hw[src] + e) * jnp.repeat(alpha, HD, axis=1)
        agg = jax.ops.segment_sum(msg, dst, num_segments=n)

        h = _ffn(h, agg, p)

    ones = jnp.ones((n,), jnp.float32)
    cnt = jax.ops.segment_sum(ones, batch, num_segments=NUM_GRAPHS)
    h_mean = jax.ops.segment_sum(h, batch, num_segments=NUM_GRAPHS) / jnp.maximum(cnt, 1.0)[:, None]
    h_max = jax.ops.segment_max(h, batch, num_segments=NUM_GRAPHS)
    h_max = jnp.where(jnp.isfinite(h_max), h_max, 0.0)
    g = jnp.concatenate([h_mean, h_max], axis=-1)
    return _final_ln(g, params['ro_s'].reshape(1, 2 * HID), params['ro_b'].reshape(1, 2 * HID))
